# baseline (speedup 1.0000x reference)
import functools

import jax
import jax.numpy as jnp
from jax import lax
from jax.experimental import pallas as pl
from jax.experimental.pallas import tpu as pltpu
from jax.experimental.pallas import tpu_sc as plsc

N_SC = 2
N_TILES = 16
K_SUP = 400


@functools.lru_cache(maxsize=None)
def _make_propagate(N, S, D):
    SPW = S // (N_SC * N_TILES)
    assert SPW % 2 == 0 and (SPW - 2) % 4 == 0
    mesh = plsc.VectorSubcoreMesh(core_axis_name="c", subcore_axis_name="s")
    STRIPE = ((N // N_TILES) + 7) // 8 * 8
    TAIL = N - (N_TILES - 1) * STRIPE

    @functools.partial(
        pl.kernel,
        mesh=mesh,
        out_type=jax.ShapeDtypeStruct((N_SC, N, D), jnp.float32),
        scratch_types=[
            pltpu.VMEM((4, K_SUP), jnp.int32),
            pltpu.VMEM((4, K_SUP), jnp.int32),
            pltpu.VMEM((2, K_SUP, D), jnp.float32),
            pltpu.VMEM_SHARED((N, D), jnp.float32),
            pltpu.SemaphoreType.DMA,
            pltpu.SemaphoreType.DMA,
            pltpu.SemaphoreType.DMA,
            pltpu.SemaphoreType.DMA,
            pltpu.SemaphoreType.DMA,
            pltpu.SemaphoreType.DMA,
            pltpu.SemaphoreType.DMA,
            pltpu.SemaphoreType.DMA,
        ],
        compiler_params=pltpu.CompilerParams(use_tc_tiling_on_sc=False),
    )
    def prop(x_hbm, src_hbm, dst_hbm, zeros_hbm, out_hbm,
             isrc, idst, rows, acc, sem_i0, sem_i1, sem_i2, sem_i3,
             sem_g0, sem_g1, sem_s0, sem_s1):
        cid = lax.axis_index("c")
        sid = lax.axis_index("s")
        r0 = sid * STRIPE

        @pl.when(sid < N_TILES - 1)
        def _():
            pltpu.sync_copy(zeros_hbm.at[pl.ds(r0, STRIPE)],
                            acc.at[pl.ds(r0, STRIPE)])

        @pl.when(sid == N_TILES - 1)
        def _():
            pltpu.sync_copy(zeros_hbm.at[pl.ds(r0, TAIL)],
                            acc.at[pl.ds(r0, TAIL)])

        plsc.subcore_barrier()

        base = (cid * N_TILES + sid) * SPW
        sem_i = (sem_i0, sem_i1, sem_i2, sem_i3)
        sem_g = (sem_g0, sem_g1)
        sem_s = (sem_s0, sem_s1)

        def idx_load(j, r):
            jj = jnp.minimum(base + j, S - 1)
            pltpu.async_copy(src_hbm.at[jj], isrc.at[r], sem_i[r])
            pltpu.async_copy(dst_hbm.at[jj], idst.at[r], sem_i[r])

        def idx_wait(r):
            pltpu.make_async_copy(src_hbm.at[base], isrc.at[r],
                                  sem_i[r]).wait()
            pltpu.make_async_copy(dst_hbm.at[base], idst.at[r],
                                  sem_i[r]).wait()

        def gather(b, r):
            pltpu.async_copy(x_hbm.at[isrc.at[r]], rows.at[b], sem_g[b])

        def gather_wait(b):
            pltpu.make_async_copy(x_hbm.at[isrc.at[0]], rows.at[b],
                                  sem_g[b]).wait()

        def scatter(b, r):
            pltpu.async_copy(rows.at[b], acc.at[idst.at[r]], sem_s[b],
                             add=True)

        def scatter_wait(b):
            pltpu.make_async_copy(rows.at[b], acc.at[idst.at[0]],
                                  sem_s[b]).wait()

        def step(j, b, ri, ri1, ri3, first=False):
            idx_wait(ri1)
            if not first:
                scatter_wait(1 - b)
            idx_load(j + 3, ri3)
            gather_wait(b)
            scatter(b, ri)
            gather(1 - b, ri1)

        pltpu.sync_copy(src_hbm.at[base], isrc.at[0])
        pltpu.sync_copy(dst_hbm.at[base], idst.at[0])
        gather(0, 0)
        idx_load(1, 1)
        idx_load(2, 2)
        step(0, 0, 0, 1, 3, first=True)
        step(1, 1, 1, 2, 0)

        def body(m, carry):
            j0 = 4 * m + 2
            for k in range(4):
                step(j0 + k, k % 2, (2 + k) % 4, (3 + k) % 4, (1 + k) % 4)
            return carry

        lax.fori_loop(0, (SPW - 2) // 4, body, 0)
        idx_wait((SPW + 1) % 4)
        idx_wait((SPW + 2) % 4)
        scatter_wait((SPW - 1) % 2)
        gather_wait(SPW % 2)
        plsc.subcore_barrier()

        @pl.when(sid < N_TILES - 1)
        def _():
            pltpu.sync_copy(acc.at[pl.ds(r0, STRIPE)],
                            out_hbm.at[cid].at[pl.ds(r0, STRIPE)])

        @pl.when(sid == N_TILES - 1)
        def _():
            pltpu.sync_copy(acc.at[pl.ds(r0, TAIL)],
                            out_hbm.at[cid].at[pl.ds(r0, TAIL)])

    return prop


_GRID = 10


def _bn(features, gamma, beta):
    N, F = features.shape
    Bn = N // _GRID

    def stats_body(x_ref, o_ref):
        @pl.when(pl.program_id(0) == 0)
        def _():
            o_ref[...] = jnp.zeros_like(o_ref)

        x = x_ref[...]
        o_ref[0, :] += jnp.sum(x, axis=0)
        o_ref[1, :] += jnp.sum(x * x, axis=0)

    stats = pl.pallas_call(
        stats_body,
        grid=(_GRID,),
        in_specs=[pl.BlockSpec((Bn, F), lambda i: (i, 0))],
        out_specs=pl.BlockSpec((2, F), lambda i: (0, 0)),
        out_shape=jax.ShapeDtypeStruct((2, F), jnp.float32),
    )(features)

    def apply_body(x_ref, s_ref, g_ref, b_ref, o_ref):
        mean = s_ref[0, :] / N
        var = s_ref[1, :] / N - mean * mean
        scale = g_ref[0, :] * lax.rsqrt(var + 1e-5)
        feat = x_ref[...] * scale + (b_ref[0, :] - mean * scale)
        o_ref[...] = jnp.concatenate(
            [feat, jnp.zeros((Bn, 16 - F), jnp.float32)], axis=1)

    return pl.pallas_call(
        apply_body,
        grid=(_GRID,),
        in_specs=[
            pl.BlockSpec((Bn, F), lambda i: (i, 0)),
            pl.BlockSpec((2, F), lambda i: (0, 0)),
            pl.BlockSpec((1, F), lambda i: (0, 0)),
            pl.BlockSpec((1, F), lambda i: (0, 0)),
        ],
        out_specs=pl.BlockSpec((Bn, 16), lambda i: (i, 0)),
        out_shape=jax.ShapeDtypeStruct((N, 16), jnp.float32),
    )(features, stats, gamma.reshape(1, -1), beta.reshape(1, -1))


def _layer1(p1, W1, b1):
    N = p1.shape[1]
    Bn = N // _GRID

    def body(p_ref, w_ref, b_ref, oa_ref, ob_ref):
        agg = (p_ref[0] + p_ref[1])[:, :4]
        y = jnp.maximum(
            jnp.dot(agg, w_ref[...], preferred_element_type=jnp.float32)
            + b_ref[...], 0.0)
        oa_ref[...] = y[:, :16]
        ob_ref[...] = y[:, 16:]

    return pl.pallas_call(
        body,
        grid=(_GRID,),
        in_specs=[
            pl.BlockSpec((N_SC, Bn, 16), lambda i: (0, i, 0)),
            pl.BlockSpec((4, 32), lambda i: (0, 0)),
            pl.BlockSpec((1, 32), lambda i: (0, 0)),
        ],
        out_specs=[
            pl.BlockSpec((Bn, 16), lambda i: (i, 0)),
            pl.BlockSpec((Bn, 16), lambda i: (i, 0)),
        ],
        out_shape=[
            jax.ShapeDtypeStruct((N, 16), jnp.float32),
            jax.ShapeDtypeStruct((N, 16), jnp.float32),
        ],
    )(p1, W1, b1.reshape(1, -1))


def _layer2_premul3(p2a, p2b, W2, b2, W3):
    N = p2a.shape[1]
    Bn = N // _GRID

    def body(pa_ref, pb_ref, w2_ref, b2_ref, w3_ref, o_ref):
        agg = jnp.concatenate(
            [pa_ref[0] + pa_ref[1], pb_ref[0] + pb_ref[1]], axis=1)
        x2 = jnp.maximum(
            jnp.dot(agg, w2_ref[...], preferred_element_type=jnp.float32)
            + b2_ref[...], 0.0)
        z = jnp.dot(x2, w3_ref[...], preferred_element_type=jnp.float32)
        o_ref[...] = jnp.concatenate(
            [z, jnp.zeros((Bn, 16 - z.shape[1]), jnp.float32)], axis=1)

    return pl.pallas_call(
        body,
        grid=(_GRID,),
        in_specs=[
            pl.BlockSpec((N_SC, Bn, 16), lambda i: (0, i, 0)),
            pl.BlockSpec((N_SC, Bn, 16), lambda i: (0, i, 0)),
            pl.BlockSpec((32, 32), lambda i: (0, 0)),
            pl.BlockSpec((1, 32), lambda i: (0, 0)),
            pl.BlockSpec((32, 5), lambda i: (0, 0)),
        ],
        out_specs=pl.BlockSpec((Bn, 16), lambda i: (i, 0)),
        out_shape=jax.ShapeDtypeStruct((N, 16), jnp.float32),
    )(p2a, p2b, W2, b2.reshape(1, -1), W3)


def _layer3_out(p3, b3):
    N = p3.shape[1]
    Bn = N // _GRID

    def body(p_ref, b_ref, o_ref):
        o_ref[...] = (p_ref[0] + p_ref[1])[:, :5] + b_ref[...]

    return pl.pallas_call(
        body,
        grid=(_GRID,),
        in_specs=[
            pl.BlockSpec((N_SC, Bn, 16), lambda i: (0, i, 0)),
            pl.BlockSpec((1, 5), lambda i: (0, 0)),
        ],
        out_specs=pl.BlockSpec((Bn, 5), lambda i: (i, 0)),
        out_shape=jax.ShapeDtypeStruct((N, 5), jnp.float32),
    )(p3, b3.reshape(1, -1))


def kernel(features, edge_index, bn_gamma, bn_beta, W1, b1, W2, b2, W3, b3):
    N = features.shape[0]
    E = edge_index.shape[1]
    S = E // K_SUP
    src2 = edge_index[0].astype(jnp.int32).reshape(S, K_SUP)
    dst2 = edge_index[1].astype(jnp.int32).reshape(S, K_SUP)

    feat = _bn(features, bn_gamma, bn_beta)
    prop16 = _make_propagate(N, S, 16)
    zeros16 = jnp.zeros((N, 16), jnp.float32)
    p1 = prop16(feat, src2, dst2, zeros16)
    x1a, x1b = _layer1(p1, W1, b1)
    p2a = prop16(x1a, src2, dst2, zeros16)
    p2b = prop16(x1b, src2, dst2, zeros16)
    z3 = _layer2_premul3(p2a, p2b, W2, b2, W3)
    p3 = prop16(z3, src2, dst2, zeros16)
    return _layer3_out(p3, b3)


# speedup vs baseline: 1.0605x; 1.0605x over previous
import functools

import jax
import jax.numpy as jnp
from jax import lax
from jax.experimental import pallas as pl
from jax.experimental.pallas import tpu as pltpu
from jax.experimental.pallas import tpu_sc as plsc

N_SC = 2
N_TILES = 16
K_SUP = 400
D = 16


@functools.lru_cache(maxsize=None)
def _make_propagate(N, S):
    SPW = S // (N_SC * N_TILES)
    assert SPW % 2 == 0
    mesh = plsc.VectorSubcoreMesh(core_axis_name="c", subcore_axis_name="s")
    STRIPE = ((N // N_TILES) + 7) // 8 * 8
    TAIL = N - (N_TILES - 1) * STRIPE

    @functools.partial(
        pl.kernel,
        mesh=mesh,
        out_type=jax.ShapeDtypeStruct((N_SC, N, D), jnp.float32),
        scratch_types=[
            pltpu.VMEM((2, K_SUP), jnp.int32),
            pltpu.VMEM((2, K_SUP), jnp.int32),
            pltpu.VMEM((2, K_SUP, D), jnp.float32),
            pltpu.VMEM_SHARED((N, D), jnp.float32),
            pltpu.SemaphoreType.DMA,
            pltpu.SemaphoreType.DMA,
            pltpu.SemaphoreType.DMA,
            pltpu.SemaphoreType.DMA,
        ],
        compiler_params=pltpu.CompilerParams(use_tc_tiling_on_sc=False),
    )
    def prop(x_hbm, src_hbm, dst_hbm, zeros_hbm, out_hbm,
             isrc, idst, rows, acc, sem_i0, sem_i1, sem_g0, sem_g1):
        cid = lax.axis_index("c")
        sid = lax.axis_index("s")
        r0 = sid * STRIPE

        @pl.when(sid < N_TILES - 1)
        def _():
            pltpu.sync_copy(zeros_hbm.at[pl.ds(r0, STRIPE)],
                            acc.at[pl.ds(r0, STRIPE)])

        @pl.when(sid == N_TILES - 1)
        def _():
            pltpu.sync_copy(zeros_hbm.at[pl.ds(r0, TAIL)],
                            acc.at[pl.ds(r0, TAIL)])

        plsc.subcore_barrier()

        base = (cid * N_TILES + sid) * SPW
        sem_i = (sem_i0, sem_i1)
        sem_g = (sem_g0, sem_g1)

        def idx_load(j, b):
            jj = jnp.minimum(base + j, S - 1)
            pltpu.async_copy(src_hbm.at[jj], isrc.at[b], sem_i[b])
            pltpu.async_copy(dst_hbm.at[jj], idst.at[b], sem_i[b])

        def idx_wait(b):
            pltpu.make_async_copy(src_hbm.at[base], isrc.at[b],
                                  sem_i[b]).wait()
            pltpu.make_async_copy(dst_hbm.at[base], idst.at[b],
                                  sem_i[b]).wait()

        def gather(b):
            pltpu.async_copy(x_hbm.at[isrc.at[b]], rows.at[b], sem_g[b])

        def gather_wait(b):
            pltpu.make_async_copy(x_hbm.at[isrc.at[b]], rows.at[b],
                                  sem_g[b]).wait()

        pltpu.sync_copy(src_hbm.at[base], isrc.at[0])
        pltpu.sync_copy(dst_hbm.at[base], idst.at[0])
        gather(0)
        idx_load(1, 1)

        def body(m, carry):
            for b in (0, 1):
                j = 2 * m + b
                nb = 1 - b
                idx_wait(nb)
                gather(nb)
                gather_wait(b)
                pltpu.sync_copy(rows.at[b], acc.at[idst.at[b]],
                                add=True)
                idx_load(j + 2, b)
            return carry

        lax.fori_loop(0, SPW // 2, body, 0)
        idx_wait(1)
        gather_wait(0)
        plsc.subcore_barrier()

        @pl.when(sid < N_TILES - 1)
        def _():
            pltpu.sync_copy(acc.at[pl.ds(r0, STRIPE)],
                            out_hbm.at[cid].at[pl.ds(r0, STRIPE)])

        @pl.when(sid == N_TILES - 1)
        def _():
            pltpu.sync_copy(acc.at[pl.ds(r0, TAIL)],
                            out_hbm.at[cid].at[pl.ds(r0, TAIL)])

    return prop


_GRID = 10


def _bn(features, gamma, beta):
    N, F = features.shape
    Bn = N // _GRID

    def stats_body(x_ref, o_ref):
        @pl.when(pl.program_id(0) == 0)
        def _():
            o_ref[...] = jnp.zeros_like(o_ref)

        x = x_ref[...]
        o_ref[0, :] += jnp.sum(x, axis=0)
        o_ref[1, :] += jnp.sum(x * x, axis=0)

    stats = pl.pallas_call(
        stats_body,
        grid=(_GRID,),
        in_specs=[pl.BlockSpec((Bn, F), lambda i: (i, 0))],
        out_specs=pl.BlockSpec((2, F), lambda i: (0, 0)),
        out_shape=jax.ShapeDtypeStruct((2, F), jnp.float32),
    )(features)

    def apply_body(x_ref, s_ref, g_ref, b_ref, o_ref):
        mean = s_ref[0, :] / N
        var = s_ref[1, :] / N - mean * mean
        scale = g_ref[0, :] * lax.rsqrt(var + 1e-5)
        feat = x_ref[...] * scale + (b_ref[0, :] - mean * scale)
        o_ref[...] = jnp.concatenate(
            [feat, jnp.zeros((Bn, D - F), jnp.float32)], axis=1)

    return pl.pallas_call(
        apply_body,
        grid=(_GRID,),
        in_specs=[
            pl.BlockSpec((Bn, F), lambda i: (i, 0)),
            pl.BlockSpec((2, F), lambda i: (0, 0)),
            pl.BlockSpec((1, F), lambda i: (0, 0)),
            pl.BlockSpec((1, F), lambda i: (0, 0)),
        ],
        out_specs=pl.BlockSpec((Bn, D), lambda i: (i, 0)),
        out_shape=jax.ShapeDtypeStruct((N, D), jnp.float32),
    )(features, stats, gamma.reshape(1, -1), beta.reshape(1, -1))


def _layer1(p1, W1, b1):
    N = p1.shape[1]
    Bn = N // _GRID

    def body(p_ref, w_ref, b_ref, oa_ref, ob_ref):
        agg = (p_ref[0] + p_ref[1])[:, :4]
        y = jnp.maximum(
            jnp.dot(agg, w_ref[...], preferred_element_type=jnp.float32)
            + b_ref[...], 0.0)
        oa_ref[...] = y[:, :16]
        ob_ref[...] = y[:, 16:]

    return pl.pallas_call(
        body,
        grid=(_GRID,),
        in_specs=[
            pl.BlockSpec((N_SC, Bn, D), lambda i: (0, i, 0)),
            pl.BlockSpec((4, 32), lambda i: (0, 0)),
            pl.BlockSpec((1, 32), lambda i: (0, 0)),
        ],
        out_specs=[
            pl.BlockSpec((Bn, 16), lambda i: (i, 0)),
            pl.BlockSpec((Bn, 16), lambda i: (i, 0)),
        ],
        out_shape=[
            jax.ShapeDtypeStruct((N, 16), jnp.float32),
            jax.ShapeDtypeStruct((N, 16), jnp.float32),
        ],
    )(p1, W1, b1.reshape(1, -1))


def _layer2_premul3(p2a, p2b, W2, b2, W3):
    N = p2a.shape[1]
    Bn = N // _GRID

    def body(pa_ref, pb_ref, w2_ref, b2_ref, w3_ref, o_ref):
        agg = jnp.concatenate(
            [pa_ref[0] + pa_ref[1], pb_ref[0] + pb_ref[1]], axis=1)
        x2 = jnp.maximum(
            jnp.dot(agg, w2_ref[...], preferred_element_type=jnp.float32)
            + b2_ref[...], 0.0)
        z = jnp.dot(x2, w3_ref[...], preferred_element_type=jnp.float32)
        o_ref[...] = jnp.concatenate(
            [z, jnp.zeros((Bn, D - z.shape[1]), jnp.float32)], axis=1)

    return pl.pallas_call(
        body,
        grid=(_GRID,),
        in_specs=[
            pl.BlockSpec((N_SC, Bn, D), lambda i: (0, i, 0)),
            pl.BlockSpec((N_SC, Bn, D), lambda i: (0, i, 0)),
            pl.BlockSpec((32, 32), lambda i: (0, 0)),
            pl.BlockSpec((1, 32), lambda i: (0, 0)),
            pl.BlockSpec((32, 5), lambda i: (0, 0)),
        ],
        out_specs=pl.BlockSpec((Bn, D), lambda i: (i, 0)),
        out_shape=jax.ShapeDtypeStruct((N, D), jnp.float32),
    )(p2a, p2b, W2, b2.reshape(1, -1), W3)


def _layer3_out(p3, b3):
    N = p3.shape[1]
    Bn = N // _GRID

    def body(p_ref, b_ref, o_ref):
        o_ref[...] = (p_ref[0] + p_ref[1])[:, :5] + b_ref[...]

    return pl.pallas_call(
        body,
        grid=(_GRID,),
        in_specs=[
            pl.BlockSpec((N_SC, Bn, D), lambda i: (0, i, 0)),
            pl.BlockSpec((1, 5), lambda i: (0, 0)),
        ],
        out_specs=pl.BlockSpec((Bn, 5), lambda i: (i, 0)),
        out_shape=jax.ShapeDtypeStruct((N, 5), jnp.float32),
    )(p3, b3.reshape(1, -1))


def kernel(features, edge_index, bn_gamma, bn_beta, W1, b1, W2, b2, W3, b3):
    N = features.shape[0]
    E = edge_index.shape[1]
    S = E // K_SUP
    src2 = edge_index[0].astype(jnp.int32).reshape(S, K_SUP)
    dst2 = edge_index[1].astype(jnp.int32).reshape(S, K_SUP)
    zeros16 = jnp.zeros((N, D), jnp.float32)
    prop = _make_propagate(N, S)

    feat = _bn(features, bn_gamma, bn_beta)
    p1 = prop(feat, src2, dst2, zeros16)
    x1a, x1b = _layer1(p1, W1, b1)
    p2a = prop(x1a, src2, dst2, zeros16)
    p2b = prop(x1b, src2, dst2, zeros16)
    z3 = _layer2_premul3(p2a, p2b, W2, b2, W3)
    p3 = prop(z3, src2, dst2, zeros16)
    return _layer3_out(p3, b3)
